# Initial kernel scaffold; baseline (speedup 1.0000x reference)
import jax
import jax.numpy as jnp
from jax.experimental import pallas as pl


def kernel(nodes, edge_index, edge_attr, valid, r, fx, loc, params):
    raise NotImplementedError("write your pallas kernel here")


# speedup vs baseline: 9.0295x; 9.0295x over previous
import functools

import jax
import jax.numpy as jnp
from jax import lax
from jax.experimental import pallas as pl
from jax.experimental.pallas import tpu as pltpu
from jax.experimental.pallas import tpu_sc as plsc

N = 100000
L = 24
E = 1600000
H = 48

FR = 19200
FPAD = FR * 128
BLK = 768
GRID_F = FR // BLK

XB = 1000
GRID_X = N // XB

EB = 12800
GRID_E = E // EB

NC, NS = 2, 16
EPT = E // NS
K = 2000
NCHUNK = EPT // K
G = 128
SEL = 2176
NBLK = SEL // G - 1
NH = N // NC
L3ROWS = 128

_RS2 = 0.7071067811865476


def _gelu(x):
    return 0.5 * x * (1.0 + lax.erf(x * _RS2))


def _mlp3s(x, w1, b1, w2, b2, w3, b3):
    h = [_gelu(x * w1[k, 0] + b1[k]) for k in range(8)]
    h2 = []
    for j in range(8):
        acc = h[0] * w2[j, 0]
        for k in range(1, 8):
            acc = acc + h[k] * w2[j, k]
        h2.append(_gelu(acc + b2[j]))
    out = h2[0] * w3[0, 0]
    for j in range(1, 8):
        out = out + h2[j] * w3[0, j]
    return out + b3[0]


_spec_s = pl.BlockSpec(memory_space=pltpu.SMEM)
_spec_f = pl.BlockSpec((BLK, 128), lambda i: (i, 0))


def _pre_body(nod_ref, val_ref, r_ref, rw1, rb1, rw2, rb2, rw3, rb3,
              rf_ref, res_ref, nv_ref):
    i = pl.program_id(0)
    v = val_ref[...]
    rf = _mlp3s(r_ref[...], rw1, rb1, rw2, rb2, rw3, rb3)
    res = nod_ref[...] * v - rf
    r0 = lax.broadcasted_iota(jnp.int32, (BLK, 128), 0)
    c0 = lax.broadcasted_iota(jnp.int32, (BLK, 128), 1)
    fl = (i * BLK + r0) * 128 + c0
    res = jnp.where(lax.rem(fl, L) == 0, 0.0, res)
    rf_ref[...] = rf
    res_ref[...] = res
    nv_ref[...] = 1.0 - v


_pre = pl.pallas_call(
    _pre_body, grid=(GRID_F,),
    in_specs=[_spec_f] * 3 + [_spec_s] * 6,
    out_specs=[_spec_f] * 3,
    out_shape=[jax.ShapeDtypeStruct((FR, 128), jnp.float32)] * 3,
)


def _edge_body(ea_ref, w1, b1, w2, b2, gw, gb, wm, wmb, out_ref):
    x = ea_ref[...]
    dot = lambda a, b: lax.dot_general(a, b, (((1,), (0,)), ((), ())),
                                       preferred_element_type=jnp.float32)
    h = _gelu(dot(w1[...], x) + b1[...])
    h = jnp.maximum(dot(w2[...], h) + b2[...], 0.0)
    g = _gelu(dot(gw[...], h) + gb[...])
    w = dot(wm[...], g) + wmb[...]
    out_ref[...] = w.reshape(1, 1, EB)


_edge = pl.pallas_call(
    _edge_body, grid=(GRID_E,),
    in_specs=[pl.BlockSpec((17, EB), lambda i: (0, i)),
              pl.BlockSpec((H, 17), lambda i: (0, 0)),
              pl.BlockSpec((H, 1), lambda i: (0, 0)),
              pl.BlockSpec((H, H), lambda i: (0, 0)),
              pl.BlockSpec((H, 1), lambda i: (0, 0)),
              pl.BlockSpec((H, H), lambda i: (0, 0)),
              pl.BlockSpec((H, 1), lambda i: (0, 0)),
              pl.BlockSpec((1, H), lambda i: (0, 0)),
              pl.BlockSpec((1, 1), lambda i: (0, 0))],
    out_specs=pl.BlockSpec((1, 1, EB), lambda i: (i, 0, 0)),
    out_shape=jax.ShapeDtypeStruct((GRID_E, 1, EB), jnp.float32),
)


def _make_u(add_rf, grid, blk):
    def body(*refs):
        if add_rf:
            (nrv_ref, res_ref, nv_ref, rfh_ref,
             fw, fb, rw1, rb1, rw2, rb2, rw3, rb3, out_ref) = refs
        else:
            (nrv_ref, res_ref, nv_ref,
             fw, fb, rw1, rb1, rw2, rb2, rw3, rb3, out_ref) = refs
        nrv = nrv_ref[...]
        res = res_ref[...]
        nv = nv_ref[...]
        m = jax.nn.sigmoid(fw[0, 0] * nrv + fw[0, 1] * res
                           + fw[0, 2] * nv + fb[0])
        r1 = (1.0 - m) * res + nv * m * nrv
        r1 = r1 + _mlp3s(r1, rw1, rb1, rw2, rb2, rw3, rb3)
        if add_rf:
            r1 = r1 + rfh_ref[...]
        out_ref[...] = r1

    nin = 4 if add_rf else 3
    spec = pl.BlockSpec((blk, 128), lambda i: (i, 0))
    return pl.pallas_call(
        body, grid=(grid,),
        in_specs=[spec] * nin + [_spec_s] * 8,
        out_specs=spec,
        out_shape=jax.ShapeDtypeStruct((grid * blk, 128), jnp.float32),
    )


_u_full = _make_u(False, GRID_F, BLK)
_u_head = _make_u(True, 1, 16)


def _x0_body(res_ref, v_ref, xt_ref):
    res = res_ref[...]
    vb = jnp.mean(v_ref[...], axis=1, keepdims=True)
    xt_ref[...] = jnp.concatenate(
        [res, vb, jnp.zeros((XB, 7), jnp.float32)], axis=1)


_spec_x24 = pl.BlockSpec((XB, L), lambda i: (i, 0))
_spec_x32 = pl.BlockSpec((XB, 32), lambda i: (i, 0))

_x0 = pl.pallas_call(
    _x0_body, grid=(GRID_X,),
    in_specs=[_spec_x24] * 2,
    out_specs=_spec_x32,
    out_shape=jax.ShapeDtypeStruct((N, 32), jnp.float32),
)


def _x_body(org_ref, res_ref, vp_ref, xt_ref, vc_ref):
    org = org_ref[...]
    res = res_ref[...]
    vp = vp_ref[...]
    vc = jnp.where(jnp.logical_or(org != res, vp > 0.0), 1.0, 0.0)
    vb = jnp.mean(vc, axis=1, keepdims=True)
    vc_ref[...] = vc
    xt_ref[...] = jnp.concatenate(
        [res, vb, jnp.zeros((XB, 7), jnp.float32)], axis=1)


_x = pl.pallas_call(
    _x_body, grid=(GRID_X,),
    in_specs=[_spec_x24] * 3,
    out_specs=[_spec_x32, _spec_x24],
    out_shape=[jax.ShapeDtypeStruct((N, 32), jnp.float32),
               jax.ShapeDtypeStruct((N, L), jnp.float32)],
)


def _make_sc(mode):
    full = mode == 'full'
    rows_sp = (NH if full else L3ROWS) + 8
    dump = rows_sp - 8
    mesh = plsc.VectorSubcoreMesh(core_axis_name="c", subcore_axis_name="s")

    @functools.partial(
        pl.kernel, mesh=mesh,
        compiler_params=pltpu.CompilerParams(
            needs_layout_passes=False, use_tc_tiling_on_sc=False),
        out_type=jax.ShapeDtypeStruct((N, 32), jnp.float32),
        scratch_types=[
            pltpu.VMEM((K,), jnp.int32),
            pltpu.VMEM((K,), jnp.int32),
            pltpu.VMEM((K,), jnp.float32),
            pltpu.VMEM((SEL,), jnp.int32),
            pltpu.VMEM((SEL,), jnp.float32),
            pltpu.VMEM((SEL,), jnp.int32),
            pltpu.VMEM((NBLK, G), jnp.int32),
            pltpu.VMEM((G, 32), jnp.float32),
            pltpu.VMEM_SHARED((rows_sp, 32), jnp.float32),
            pltpu.SemaphoreType.DMA,
        ],
    )
    def sc_kernel(src_hbm, dst_hbm, w_hbm, xtab_hbm, nrv_hbm,
                  srcc, dstc, wc, ssrc, swt, sdst, sdst2, rows,
                  nrv_sh, sem):
        c = lax.axis_index("c")
        s = lax.axis_index("s")

        zv = jnp.zeros((16,), jnp.float32)

        def zrow(i, _):
            rows[i, pl.ds(0, 16)] = zv
            rows[i, pl.ds(16, 16)] = zv
            return 0

        lax.fori_loop(0, G, zrow, 0)
        nzfull = rows_sp // G
        zrem = rows_sp - nzfull * G
        for t in range((nzfull + NS) // NS):
            idx = s + NS * t

            @pl.when(idx < nzfull)
            def _():
                pltpu.sync_copy(rows, nrv_sh.at[pl.ds(idx * G, G)])

            if zrem:
                @pl.when(idx == nzfull)
                def _():
                    pltpu.sync_copy(rows.at[pl.ds(0, zrem)],
                                    nrv_sh.at[pl.ds(nzfull * G, zrem)])

        plsc.subcore_barrier()

        lo = c * NH
        zi = jnp.zeros((16,), jnp.int32)
        di = jnp.full((16,), dump, jnp.int32)
        zf = jnp.zeros((16,), jnp.float32)
        iota16 = lax.iota(jnp.int32, 16)
        col24 = jnp.full((16,), 24, jnp.int32)

        def chunk(kk, _):
            base = s * EPT + kk * K
            pltpu.sync_copy(src_hbm.at[pl.ds(base, K)], srcc)
            pltpu.sync_copy(dst_hbm.at[pl.ds(base, K)], dstc)
            pltpu.sync_copy(w_hbm.at[pl.ds(base, K)], wc)

            def comp(i, off):
                o16 = i * 16
                dv = dstc[pl.ds(o16, 16)]
                srcv = srcc[pl.ds(o16, 16)]
                wv = wc[pl.ds(o16, 16)]
                if full:
                    inb = jnp.logical_and(dv >= lo, dv < lo + NH)
                    row = dv - lo
                else:
                    inb = jnp.logical_and(dv == 0,
                                          jnp.full((16,), c, jnp.int32) == 0)
                    row = dv
                pos = plsc.cumsum(inb.astype(jnp.int32))
                tgt = off + pos - 1
                plsc.store_scatter(ssrc, [tgt], srcv, mask=inb)
                plsc.store_scatter(swt, [tgt], wv, mask=inb)
                plsc.store_scatter(sdst, [tgt], row, mask=inb)
                return off + pos[15]

            cnt = lax.fori_loop(0, K // 16, comp, jnp.int32(0))

            for t in range(G // 16):
                ssrc[pl.ds(cnt + t * 16, 16)] = zi
                sdst[pl.ds(cnt + t * 16, 16)] = di
                swt[pl.ds(cnt + t * 16, 16)] = zf

            for j in range(NBLK):
                for h8 in range(G // 16):
                    sdst2[j, pl.ds(h8 * 16, 16)] = \
                        sdst[pl.ds(j * G + h8 * 16, 16)]

            nblk = (cnt + G - 1) // G

            def gs(j, _):
                jg = j * G
                pltpu.async_copy(
                    xtab_hbm.at[ssrc.at[pl.ds(jg, G)]], rows, sem).wait()
                for t in range(G // 16):
                    ridx = iota16 + t * 16
                    vb = plsc.load_gather(rows, [ridx, col24])
                    wv = swt[pl.ds(jg + t * 16, 16)]
                    sg = 1.0 / (1.0 + jnp.exp(-(vb * wv)))
                    for l in range(16):
                        rr = t * 16 + l
                        sc_ = sg[l]
                        rows[rr, pl.ds(0, 16)] = rows[rr, pl.ds(0, 16)] * sc_
                        rows[rr, pl.ds(16, 16)] = \
                            rows[rr, pl.ds(16, 16)] * sc_
                pltpu.sync_copy(rows, nrv_sh.at[sdst2.at[j]], add=True)
                return 0

            lax.fori_loop(0, nblk, gs, 0)
            return 0

        lax.fori_loop(0, NCHUNK, chunk, 0)
        plsc.subcore_barrier()

        if full:
            nwfull = NH // G
            wrem = NH - nwfull * G
            for t in range((nwfull + NS) // NS):
                idx = s + NS * t

                @pl.when(idx < nwfull)
                def _():
                    r0 = idx * G
                    pltpu.sync_copy(nrv_sh.at[pl.ds(r0, G)],
                                    nrv_hbm.at[pl.ds(c * NH + r0, G)])

                if wrem:
                    @pl.when(idx == nwfull)
                    def _():
                        r0 = nwfull * G
                        pltpu.sync_copy(nrv_sh.at[pl.ds(r0, wrem)],
                                        nrv_hbm.at[pl.ds(c * NH + r0, wrem)])
        else:
            @pl.when(jnp.logical_and(c == 0, s == 0))
            def _():
                pltpu.sync_copy(nrv_sh.at[pl.ds(0, G)],
                                nrv_hbm.at[pl.ds(0, G)])

    return sc_kernel


_sc_full = _make_sc('full')
_sc_l3 = _make_sc('l3')


def kernel(nodes, edge_index, edge_attr, valid, r, fx, loc, params):
    p = params
    v = valid[0]
    rr = r[0]
    nod = nodes[..., 0]

    def flat(a):
        return jnp.pad(a.reshape(-1), (0, FPAD - N * L)).reshape(FR, 128)

    def unflat(a):
        return a.reshape(-1)[:N * L].reshape(N, L)

    rain_w = (p['rain_W1'], p['rain_b1'], p['rain_W2'], p['rain_b2'],
              p['rain_W3'], p['rain_b3'])
    res_w = (p['res_W1'], p['res_b1'], p['res_W2'], p['res_b2'],
             p['res_W3'], p['res_b3'])

    rf_f, res0_f, nv_f = _pre(flat(nod), flat(v), flat(rr), *rain_w)

    w3 = _edge(edge_attr.T,
               p['le_W1'], p['le_b1'].reshape(H, 1),
               p['le_W2'], p['le_b2'].reshape(H, 1),
               p['gru_W'], p['gru_b'].reshape(H, 1),
               p['wm_W'], p['wm_b'].reshape(1, 1))
    w_e = w3.reshape(E)

    src = edge_index[0]
    dst = edge_index[1]
    res0 = unflat(res0_f)
    orig = res0
    xtab = _x0(res0, v)
    res_f = res0_f
    vcur = v

    for _layer in (1, 2):
        nrv = _sc_full(src, dst, w_e, xtab)
        nrv24_f = flat(nrv[:, :L])
        res_f = _u_full(nrv24_f, res_f, nv_f,
                        p['fuse_W'], p['fuse_b'], *res_w)
        xtab, vcur = _x(orig, unflat(res_f), vcur)

    nrv = _sc_l3(src, dst, w_e, xtab)
    nrvh = nrv[:L3ROWS, :L].reshape(-1)[:2048].reshape(16, 128)
    res3h = _u_head(nrvh, res_f[:16], nv_f[:16], rf_f[:16],
                    p['fuse_W'], p['fuse_b'], *res_w)
    out0 = res3h.reshape(-1)[:L][None, :]
    rf0 = rf_f.reshape(-1)[:L][None, :]
    return (out0, rf0)
